# baseline (speedup 1.0000x reference)
import functools

import jax
import jax.numpy as jnp
from jax import lax
from jax.experimental import pallas as pl
from jax.experimental.pallas import tpu as pltpu
from jax.experimental.pallas import tpu_sc as plsc

K = 16
RADIUS = 0.05
NSEG = 256


def _rtne_bf16(v):
    u = lax.bitcast_convert_type(v, jnp.uint32)
    r = (u + jnp.uint32(0x7FFF) + ((u >> 16) & jnp.uint32(1))) \
        & jnp.uint32(0xFFFF0000)
    return lax.bitcast_convert_type(r, jnp.float32)


def _stage_a(xyz1_ref, xyz2t_ref, r1_ref, r2t_ref,
             segmin_ref, segarg_ref, sq1_ref, sq2_ref, *, n2):
    q = xyz1_ref.shape[1]
    p1 = xyz1_ref[0]
    sq1 = (p1[:, 0:1] * p1[:, 0:1]
           + p1[:, 1:2] * p1[:, 1:2]
           + p1[:, 2:3] * p1[:, 2:3])
    p2 = xyz2t_ref[0]
    sq2 = (p2[0:1, :] * p2[0:1, :]
           + p2[1:2, :] * p2[1:2, :]
           + p2[2:3, :] * p2[2:3, :])
    b1 = r1_ref[0]
    b2 = r2t_ref[0]
    inner = (b1[:, 0:1] * b2[0:1, :]
             + b1[:, 1:2] * b2[1:2, :]
             + b1[:, 2:3] * b2[2:3, :])
    d2 = (sq1 + sq2) - 2.0 * inner
    d2 = jnp.maximum(d2, 0.0)

    acc = d2[:, 0:NSEG]
    member = jnp.zeros((q, NSEG), jnp.int32)
    for r in range(1, n2 // NSEG):
        sl = d2[:, r * NSEG:(r + 1) * NSEG]
        upd = sl < acc
        acc = jnp.where(upd, sl, acc)
        member = jnp.where(upd, r, member)
    lane = lax.broadcasted_iota(jnp.int32, (q, NSEG), 1)
    segmin_ref[0] = acc
    segarg_ref[0] = member * NSEG + lane
    sq1_ref[0] = sq1
    sq2_ref[0] = sq2


def _make_sc_topk(nrows, rpw, rblk, n2, n1):
    mesh = plsc.VectorSubcoreMesh(core_axis_name="c", subcore_axis_name="s",
                                  num_cores=2)

    @functools.partial(
        pl.kernel,
        out_type=[jax.ShapeDtypeStruct((nrows * K,), jnp.float32),
                  jax.ShapeDtypeStruct((nrows * K,), jnp.int32)],
        mesh=mesh,
        compiler_params=pltpu.CompilerParams(needs_layout_passes=False),
        scratch_types=[
            pltpu.VMEM((4 * n2,), jnp.float32),
            pltpu.VMEM((rblk * NSEG,), jnp.float32),
            pltpu.VMEM((rblk * NSEG,), jnp.int32),
            pltpu.VMEM((rblk * 4,), jnp.float32),
            pltpu.VMEM((rblk * K,), jnp.float32),
            pltpu.VMEM((rblk * K,), jnp.int32),
        ],
    )
    def knn_sc(segmin, segarg, kpack, qpack, dist, idx,
               keys_v, seg_v, arg_v, q_v, od_v, oi_v):
        wid = lax.axis_index("s") * 2 + lax.axis_index("c")
        base0 = wid * rpw
        batch = base0 // n1
        pltpu.sync_copy(kpack.at[pl.ds(batch * 4 * n2, 4 * n2)], keys_v)
        zero16 = jnp.zeros((K,), jnp.int32)
        iota16 = lax.broadcasted_iota(jnp.int32, (K,), 0)
        par0 = jnp.bitwise_xor(iota16, 1)
        par1 = jnp.clip(iota16 + jnp.where((iota16 & 1) > 0, 1, -1), 0, K - 1)

        def shuf(x, p):
            return x.at[p].get(mode="promise_in_bounds")

        def lexless(ak, av, bk, bv):
            return (ak < bk) | ((ak == bk) & (av < bv))

        def fix_pass(k, v, p):
            pk = shuf(k, p)
            pv = shuf(v, p)
            partner_less = lexless(pk, pv, k, v)
            self_less = lexless(k, v, pk, pv)
            sel = jnp.where(iota16 < p, partner_less, self_less)
            return jnp.where(sel, pk, k), jnp.where(sel, pv, v)

        def srt(k, v):
            ks, vs = plsc.sort_key_val(k, v)
            ks, vs = fix_pass(ks, vs, par0)
            ks, vs = fix_pass(ks, vs, par1)
            return ks, vs

        def srt_fast(k, v):
            return plsc.sort_key_val(k, v)

        def merge_with(sort_fn, rk, rv, ck, cv):
            bk = lax.rev(rk, (0,))
            bv = lax.rev(rv, (0,))
            take = lexless(ck, cv, bk, bv)
            mk = jnp.where(take, ck, bk)
            mv = jnp.where(take, cv, bv)
            return sort_fn(mk, mv)

        def merge(rk, rv, ck, cv):
            return merge_with(srt, rk, rv, ck, cv)

        def merge_fast(rk, rv, ck, cv):
            return merge_with(srt_fast, rk, rv, ck, cv)

        def row_body(r):
            q4 = jnp.full((K,), r * 4, jnp.int32)
            qx = plsc.load_gather(q_v, [q4])
            qy = plsc.load_gather(q_v, [q4 + 1])
            qz = plsc.load_gather(q_v, [q4 + 2])
            s1 = plsc.load_gather(q_v, [q4 + 3])
            sbase = jnp.full((K,), r * NSEG, jnp.int32) + iota16
            lists = []
            for c in range(NSEG // K):
                sv = plsc.load_gather(seg_v, [sbase + c * K])
                si = plsc.load_gather(arg_v, [sbase + c * K])
                lists.append(srt_fast(sv, si))
            while len(lists) > 1:
                lists = [merge_fast(a[0], a[1], b[0], b[1])
                         for a, b in zip(lists[::2], lists[1::2])]
            rk, rv = lists[0]
            segc = rv & (NSEG - 1)
            lists = []
            for m in range(n2 // NSEG):
                kidx = segc + m * NSEG
                gx = plsc.load_gather(keys_v, [kidx])
                gy = plsc.load_gather(keys_v, [kidx + n2])
                gz = plsc.load_gather(keys_v, [kidx + 2 * n2])
                g2 = plsc.load_gather(keys_v, [kidx + 3 * n2])
                inner = qx * gx + qy * gy + qz * gz
                raw = (s1 + g2) - 2.0 * inner
                zk = (kidx.astype(jnp.float32) - float(n2)) * 1e-10
                skey = jnp.where(raw > 0.0, raw, zk)
                lists.append(srt(skey, kidx))
            while len(lists) > 1:
                lists = [merge(a[0], a[1], b[0], b[1])
                         for a, b in zip(lists[::2], lists[1::2])]
            dk, dv = lists[0]
            obase = jnp.full((K,), r * K, jnp.int32) + iota16
            plsc.store_scatter(od_v, [obase], jnp.maximum(dk, 0.0))
            plsc.store_scatter(oi_v, [obase], dv)

        def do_pair(i, _):
            row_body(i * 2)
            row_body(i * 2 + 1)
            return 0

        def do_block(blk, _):
            rowbase = base0 + blk * rblk
            pltpu.sync_copy(segmin.at[pl.ds(rowbase * NSEG, rblk * NSEG)],
                            seg_v)
            pltpu.sync_copy(segarg.at[pl.ds(rowbase * NSEG, rblk * NSEG)],
                            arg_v)
            pltpu.sync_copy(qpack.at[pl.ds(rowbase * 4, rblk * 4)], q_v)
            lax.fori_loop(0, rblk // 2, do_pair, 0)
            pltpu.sync_copy(od_v, dist.at[pl.ds(rowbase * K, rblk * K)])
            pltpu.sync_copy(oi_v, idx.at[pl.ds(rowbase * K, rblk * K)])
            return 0

        lax.fori_loop(0, rpw // rblk, do_block, 0)

    return knn_sc


def kernel(xyz1, xyz2):
    b, n1, _ = xyz1.shape
    n2 = xyz2.shape[1]
    q = 512
    xyz2t = jnp.transpose(xyz2, (0, 2, 1))
    r1 = _rtne_bf16(xyz1)
    r2t = _rtne_bf16(xyz2t)

    segmin, segarg, sq1, sq2 = pl.pallas_call(
        functools.partial(_stage_a, n2=n2),
        grid=(b, n1 // q),
        in_specs=[
            pl.BlockSpec((1, q, 3), lambda i, j: (i, j, 0)),
            pl.BlockSpec((1, 3, n2), lambda i, j: (i, 0, 0)),
            pl.BlockSpec((1, q, 3), lambda i, j: (i, j, 0)),
            pl.BlockSpec((1, 3, n2), lambda i, j: (i, 0, 0)),
        ],
        out_specs=[
            pl.BlockSpec((1, q, NSEG), lambda i, j: (i, j, 0)),
            pl.BlockSpec((1, q, NSEG), lambda i, j: (i, j, 0)),
            pl.BlockSpec((1, q, 1), lambda i, j: (i, j, 0)),
            pl.BlockSpec((1, 1, n2), lambda i, j: (i, 0, 0)),
        ],
        out_shape=[
            jax.ShapeDtypeStruct((b, n1, NSEG), jnp.float32),
            jax.ShapeDtypeStruct((b, n1, NSEG), jnp.int32),
            jax.ShapeDtypeStruct((b, n1, 1), jnp.float32),
            jax.ShapeDtypeStruct((b, 1, n2), jnp.float32),
        ],
    )(xyz1, xyz2t, r1, r2t)

    nrows = b * n1
    kpack = jnp.concatenate([r2t, sq2], axis=1).reshape(b * 4 * n2)
    qpack = jnp.concatenate([r1, sq1], axis=2).reshape(nrows * 4)
    segmin_flat = segmin.reshape(nrows * NSEG)
    segarg_flat = segarg.reshape(nrows * NSEG)

    nworkers = 32
    rpw = nrows // nworkers
    rblk = 128
    dist_flat, idx_flat = _make_sc_topk(nrows, rpw, rblk, n2, n1)(
        segmin_flat, segarg_flat, kpack, qpack)
    dist = dist_flat.reshape(b, n1, K)
    idx = idx_flat.reshape(b, n1, K)
    mask = dist <= RADIUS
    return dist, idx, mask


# speedup vs baseline: 1.1439x; 1.1439x over previous
import functools

import jax
import jax.numpy as jnp
from jax import lax
from jax.experimental import pallas as pl
from jax.experimental.pallas import tpu as pltpu
from jax.experimental.pallas import tpu_sc as plsc

K = 16
RADIUS = 0.05
NSEG = 256


def _rtne_bf16(v):
    u = lax.bitcast_convert_type(v, jnp.uint32)
    r = (u + jnp.uint32(0x7FFF) + ((u >> 16) & jnp.uint32(1))) \
        & jnp.uint32(0xFFFF0000)
    return lax.bitcast_convert_type(r, jnp.float32)


def _stage_a(xyz1_ref, xyz2t_ref, r1_ref, r2t_ref,
             segmin_ref, segarg_ref, sq1_ref, sq2_ref, *, n2):
    q = xyz1_ref.shape[1]
    p1 = xyz1_ref[0]
    sq1 = (p1[:, 0:1] * p1[:, 0:1]
           + p1[:, 1:2] * p1[:, 1:2]
           + p1[:, 2:3] * p1[:, 2:3])
    p2 = xyz2t_ref[0]
    sq2 = (p2[0:1, :] * p2[0:1, :]
           + p2[1:2, :] * p2[1:2, :]
           + p2[2:3, :] * p2[2:3, :])
    b1 = r1_ref[0]
    b2 = r2t_ref[0]
    inner = jnp.dot(b1, b2, preferred_element_type=jnp.float32)
    d2 = (sq1 + sq2) - 2.0 * inner

    acc = d2[:, 0:NSEG]
    member = jnp.zeros((q, NSEG), jnp.int32)
    for r in range(1, n2 // NSEG):
        sl = d2[:, r * NSEG:(r + 1) * NSEG]
        upd = sl < acc
        acc = jnp.where(upd, sl, acc)
        member = jnp.where(upd, r, member)
    lane = lax.broadcasted_iota(jnp.int32, (q, NSEG), 1)
    segmin_ref[0] = jnp.maximum(acc, 0.0)
    segarg_ref[0] = member * NSEG + lane
    sq1_ref[0] = sq1
    sq2_ref[0] = sq2


def _make_sc_topk(nrows, rpw, rblk, n2, n1):
    mesh = plsc.VectorSubcoreMesh(core_axis_name="c", subcore_axis_name="s",
                                  num_cores=2)

    @functools.partial(
        pl.kernel,
        out_type=[jax.ShapeDtypeStruct((nrows * K,), jnp.float32),
                  jax.ShapeDtypeStruct((nrows * K,), jnp.int32)],
        mesh=mesh,
        compiler_params=pltpu.CompilerParams(needs_layout_passes=False),
        scratch_types=[
            pltpu.VMEM((4 * n2,), jnp.float32),
            pltpu.VMEM((rblk * NSEG,), jnp.float32),
            pltpu.VMEM((rblk * NSEG,), jnp.int32),
            pltpu.VMEM((rblk * 4,), jnp.float32),
            pltpu.VMEM((rblk * K,), jnp.float32),
            pltpu.VMEM((rblk * K,), jnp.int32),
        ],
    )
    def knn_sc(segmin, segarg, kpack, qpack, dist, idx,
               keys_v, seg_v, arg_v, q_v, od_v, oi_v):
        wid = lax.axis_index("s") * 2 + lax.axis_index("c")
        base0 = wid * rpw
        batch = base0 // n1
        pltpu.sync_copy(kpack.at[pl.ds(batch * 4 * n2, 4 * n2)], keys_v)
        zero16 = jnp.zeros((K,), jnp.int32)
        iota16 = lax.broadcasted_iota(jnp.int32, (K,), 0)
        par0 = jnp.bitwise_xor(iota16, 1)
        par1 = jnp.clip(iota16 + jnp.where((iota16 & 1) > 0, 1, -1), 0, K - 1)

        def shuf(x, p):
            return x.at[p].get(mode="promise_in_bounds")

        def lexless(ak, av, bk, bv):
            return (ak < bk) | ((ak == bk) & (av < bv))

        def fix_pass(k, v, p):
            pk = shuf(k, p)
            pv = shuf(v, p)
            partner_less = lexless(pk, pv, k, v)
            self_less = lexless(k, v, pk, pv)
            sel = jnp.where(iota16 < p, partner_less, self_less)
            return jnp.where(sel, pk, k), jnp.where(sel, pv, v)

        def srt(k, v):
            ks, vs = plsc.sort_key_val(k, v)
            ks, vs = fix_pass(ks, vs, par0)
            ks, vs = fix_pass(ks, vs, par1)
            return ks, vs

        def srt_fast(k, v):
            return plsc.sort_key_val(k, v)

        def merge_with(sort_fn, rk, rv, ck, cv):
            bk = lax.rev(rk, (0,))
            bv = lax.rev(rv, (0,))
            take = lexless(ck, cv, bk, bv)
            mk = jnp.where(take, ck, bk)
            mv = jnp.where(take, cv, bv)
            return sort_fn(mk, mv)

        def merge(rk, rv, ck, cv):
            return merge_with(srt, rk, rv, ck, cv)

        def merge_fast(rk, rv, ck, cv):
            return merge_with(srt_fast, rk, rv, ck, cv)

        def row_body(r):
            q4 = jnp.full((K,), r * 4, jnp.int32)
            qx = plsc.load_gather(q_v, [q4])
            qy = plsc.load_gather(q_v, [q4 + 1])
            qz = plsc.load_gather(q_v, [q4 + 2])
            s1 = plsc.load_gather(q_v, [q4 + 3])
            sbase = jnp.full((K,), r * NSEG, jnp.int32) + iota16
            lists = []
            for c in range(NSEG // K):
                sv = plsc.load_gather(seg_v, [sbase + c * K])
                si = plsc.load_gather(arg_v, [sbase + c * K])
                lists.append(srt_fast(sv, si))
            while len(lists) > 1:
                lists = [merge_fast(a[0], a[1], b[0], b[1])
                         for a, b in zip(lists[::2], lists[1::2])]
            rk, rv = lists[0]
            segc = rv & (NSEG - 1)
            lists = []
            for m in range(n2 // NSEG):
                kidx = segc + m * NSEG
                gx = plsc.load_gather(keys_v, [kidx])
                gy = plsc.load_gather(keys_v, [kidx + n2])
                gz = plsc.load_gather(keys_v, [kidx + 2 * n2])
                g2 = plsc.load_gather(keys_v, [kidx + 3 * n2])
                inner = qx * gx + qy * gy + qz * gz
                raw = (s1 + g2) - 2.0 * inner
                zk = (kidx.astype(jnp.float32) - float(n2)) * 1e-10
                skey = jnp.where(raw > 0.0, raw, zk)
                lists.append(srt(skey, kidx))
            while len(lists) > 1:
                lists = [merge(a[0], a[1], b[0], b[1])
                         for a, b in zip(lists[::2], lists[1::2])]
            dk, dv = lists[0]
            obase = jnp.full((K,), r * K, jnp.int32) + iota16
            plsc.store_scatter(od_v, [obase], jnp.maximum(dk, 0.0))
            plsc.store_scatter(oi_v, [obase], dv)

        def do_pair(i, _):
            row_body(i * 2)
            row_body(i * 2 + 1)
            return 0

        def do_block(blk, _):
            rowbase = base0 + blk * rblk
            pltpu.sync_copy(segmin.at[pl.ds(rowbase * NSEG, rblk * NSEG)],
                            seg_v)
            pltpu.sync_copy(segarg.at[pl.ds(rowbase * NSEG, rblk * NSEG)],
                            arg_v)
            pltpu.sync_copy(qpack.at[pl.ds(rowbase * 4, rblk * 4)], q_v)
            lax.fori_loop(0, rblk // 2, do_pair, 0)
            pltpu.sync_copy(od_v, dist.at[pl.ds(rowbase * K, rblk * K)])
            pltpu.sync_copy(oi_v, idx.at[pl.ds(rowbase * K, rblk * K)])
            return 0

        lax.fori_loop(0, rpw // rblk, do_block, 0)

    return knn_sc


def kernel(xyz1, xyz2):
    b, n1, _ = xyz1.shape
    n2 = xyz2.shape[1]
    q = 512
    xyz2t = jnp.transpose(xyz2, (0, 2, 1))
    r1 = _rtne_bf16(xyz1)
    r2t = _rtne_bf16(xyz2t)

    segmin, segarg, sq1, sq2 = pl.pallas_call(
        functools.partial(_stage_a, n2=n2),
        grid=(b, n1 // q),
        in_specs=[
            pl.BlockSpec((1, q, 3), lambda i, j: (i, j, 0)),
            pl.BlockSpec((1, 3, n2), lambda i, j: (i, 0, 0)),
            pl.BlockSpec((1, q, 3), lambda i, j: (i, j, 0)),
            pl.BlockSpec((1, 3, n2), lambda i, j: (i, 0, 0)),
        ],
        out_specs=[
            pl.BlockSpec((1, q, NSEG), lambda i, j: (i, j, 0)),
            pl.BlockSpec((1, q, NSEG), lambda i, j: (i, j, 0)),
            pl.BlockSpec((1, q, 1), lambda i, j: (i, j, 0)),
            pl.BlockSpec((1, 1, n2), lambda i, j: (i, 0, 0)),
        ],
        out_shape=[
            jax.ShapeDtypeStruct((b, n1, NSEG), jnp.float32),
            jax.ShapeDtypeStruct((b, n1, NSEG), jnp.int32),
            jax.ShapeDtypeStruct((b, n1, 1), jnp.float32),
            jax.ShapeDtypeStruct((b, 1, n2), jnp.float32),
        ],
    )(xyz1, xyz2t, r1, r2t)

    nrows = b * n1
    kpack = jnp.concatenate([r2t, sq2], axis=1).reshape(b * 4 * n2)
    qpack = jnp.concatenate([r1, sq1], axis=2).reshape(nrows * 4)
    segmin_flat = segmin.reshape(nrows * NSEG)
    segarg_flat = segarg.reshape(nrows * NSEG)

    nworkers = 32
    rpw = nrows // nworkers
    rblk = 128
    dist_flat, idx_flat = _make_sc_topk(nrows, rpw, rblk, n2, n1)(
        segmin_flat, segarg_flat, kpack, qpack)
    dist = dist_flat.reshape(b, n1, K)
    idx = idx_flat.reshape(b, n1, K)
    mask = dist <= RADIUS
    return dist, idx, mask


# speedup vs baseline: 1.2200x; 1.0666x over previous
import functools

import jax
import jax.numpy as jnp
from jax import lax
from jax.experimental import pallas as pl
from jax.experimental.pallas import tpu as pltpu
from jax.experimental.pallas import tpu_sc as plsc

K = 16
RADIUS = 0.05
NSEG = 256


def _rtne_bf16(v):
    u = lax.bitcast_convert_type(v, jnp.uint32)
    r = (u + jnp.uint32(0x7FFF) + ((u >> 16) & jnp.uint32(1))) \
        & jnp.uint32(0xFFFF0000)
    return lax.bitcast_convert_type(r, jnp.float32)


def _stage_a(xyz1_ref, xyz2t_ref, r1_ref, r2t_ref,
             segmin_ref, segarg_ref, sq1_ref, sq2_ref, *, n2):
    q = xyz1_ref.shape[1]
    p1 = xyz1_ref[0]
    sq1 = (p1[:, 0:1] * p1[:, 0:1]
           + p1[:, 1:2] * p1[:, 1:2]
           + p1[:, 2:3] * p1[:, 2:3])
    p2 = xyz2t_ref[0]
    sq2 = (p2[0:1, :] * p2[0:1, :]
           + p2[1:2, :] * p2[1:2, :]
           + p2[2:3, :] * p2[2:3, :])
    b1 = r1_ref[0]
    b2 = r2t_ref[0]
    inner = jnp.dot(b1, b2, preferred_element_type=jnp.float32)
    d2 = (sq1 + sq2) - 2.0 * inner

    acc = d2[:, 0:NSEG]
    member = jnp.zeros((q, NSEG), jnp.int32)
    for r in range(1, n2 // NSEG):
        sl = d2[:, r * NSEG:(r + 1) * NSEG]
        upd = sl < acc
        acc = jnp.where(upd, sl, acc)
        member = jnp.where(upd, r, member)
    lane = lax.broadcasted_iota(jnp.int32, (q, NSEG), 1)
    segmin_ref[0] = jnp.maximum(acc, 0.0)
    segarg_ref[0] = member * NSEG + lane
    sq1_ref[0] = sq1
    sq2_ref[0] = sq2


def _make_sc_topk(nrows, rpw, rblk, n2, n1):
    mesh = plsc.VectorSubcoreMesh(core_axis_name="c", subcore_axis_name="s",
                                  num_cores=2)

    @functools.partial(
        pl.kernel,
        out_type=[jax.ShapeDtypeStruct((nrows * K,), jnp.float32),
                  jax.ShapeDtypeStruct((nrows * K,), jnp.int32)],
        mesh=mesh,
        compiler_params=pltpu.CompilerParams(needs_layout_passes=False),
        scratch_types=[
            pltpu.VMEM((4 * n2,), jnp.float32),
            pltpu.VMEM((rblk * NSEG,), jnp.float32),
            pltpu.VMEM((rblk * NSEG,), jnp.int32),
            pltpu.VMEM((rblk * 4,), jnp.float32),
            pltpu.VMEM((rblk * K,), jnp.float32),
            pltpu.VMEM((rblk * K,), jnp.int32),
        ],
    )
    def knn_sc(segmin, segarg, kpack, qpack, dist, idx,
               keys_v, seg_v, arg_v, q_v, od_v, oi_v):
        wid = lax.axis_index("s") * 2 + lax.axis_index("c")
        base0 = wid * rpw
        batch = base0 // n1
        pltpu.sync_copy(kpack.at[pl.ds(batch * 4 * n2, 4 * n2)], keys_v)
        zero16 = jnp.zeros((K,), jnp.int32)
        iota16 = lax.broadcasted_iota(jnp.int32, (K,), 0)
        par0 = jnp.bitwise_xor(iota16, 1)
        par1 = jnp.clip(iota16 + jnp.where((iota16 & 1) > 0, 1, -1), 0, K - 1)

        def shuf(x, p):
            return x.at[p].get(mode="promise_in_bounds")

        def lexless(ak, av, bk, bv):
            return (ak < bk) | ((ak == bk) & (av < bv))

        def fix_pass(k, v, p):
            pk = shuf(k, p)
            pv = shuf(v, p)
            partner_less = lexless(pk, pv, k, v)
            self_less = lexless(k, v, pk, pv)
            sel = jnp.where(iota16 < p, partner_less, self_less)
            return jnp.where(sel, pk, k), jnp.where(sel, pv, v)

        def srt(k, v):
            ks, vs = plsc.sort_key_val(k, v)
            ks, vs = fix_pass(ks, vs, par0)
            ks, vs = fix_pass(ks, vs, par1)
            return ks, vs

        def srt_fast(k, v):
            return plsc.sort_key_val(k, v)

        def merge_with(sort_fn, rk, rv, ck, cv):
            bk = lax.rev(rk, (0,))
            bv = lax.rev(rv, (0,))
            take = lexless(ck, cv, bk, bv)
            mk = jnp.where(take, ck, bk)
            mv = jnp.where(take, cv, bv)
            return sort_fn(mk, mv)

        def merge(rk, rv, ck, cv):
            return merge_with(srt, rk, rv, ck, cv)

        def merge_fast(rk, rv, ck, cv):
            return merge_with(srt_fast, rk, rv, ck, cv)

        def row_body(r):
            q4 = jnp.full((K,), r * 4, jnp.int32)
            qx = plsc.load_gather(q_v, [q4])
            qy = plsc.load_gather(q_v, [q4 + 1])
            qz = plsc.load_gather(q_v, [q4 + 2])
            s1 = plsc.load_gather(q_v, [q4 + 3])
            sbase = jnp.full((K,), r * NSEG, jnp.int32) + iota16
            lists = []
            for c in range(NSEG // K):
                sv = plsc.load_gather(seg_v, [sbase + c * K])
                si = plsc.load_gather(arg_v, [sbase + c * K])
                lists.append(srt_fast(sv, si))
            while len(lists) > 1:
                lists = [merge_fast(a[0], a[1], b[0], b[1])
                         for a, b in zip(lists[::2], lists[1::2])]
            rk, rv = lists[0]
            segc = rv & (NSEG - 1)
            lists = []
            for m in range(n2 // NSEG):
                kidx = segc + m * NSEG
                gx = plsc.load_gather(keys_v, [kidx])
                gy = plsc.load_gather(keys_v, [kidx + n2])
                gz = plsc.load_gather(keys_v, [kidx + 2 * n2])
                g2 = plsc.load_gather(keys_v, [kidx + 3 * n2])
                inner = qx * gx + qy * gy + qz * gz
                raw = (s1 + g2) - 2.0 * inner
                zk = (kidx.astype(jnp.float32) - float(n2)) * 1e-10
                skey = jnp.where(raw > 0.0, raw, zk)
                lists.append(srt(skey, kidx))
            while len(lists) > 1:
                lists = [merge(a[0], a[1], b[0], b[1])
                         for a, b in zip(lists[::2], lists[1::2])]
            dk, dv = lists[0]
            obase = jnp.full((K,), r * K, jnp.int32) + iota16
            plsc.store_scatter(od_v, [obase], jnp.maximum(dk, 0.0))
            plsc.store_scatter(oi_v, [obase], dv)

        def do_pair(i, _):
            row_body(i * 2)
            row_body(i * 2 + 1)
            return 0

        def do_block(blk, _):
            rowbase = base0 + blk * rblk
            pltpu.sync_copy(segmin.at[pl.ds(rowbase * NSEG, rblk * NSEG)],
                            seg_v)
            pltpu.sync_copy(segarg.at[pl.ds(rowbase * NSEG, rblk * NSEG)],
                            arg_v)
            pltpu.sync_copy(qpack.at[pl.ds(rowbase * 4, rblk * 4)], q_v)
            lax.fori_loop(0, rblk // 2, do_pair, 0)
            pltpu.sync_copy(od_v, dist.at[pl.ds(rowbase * K, rblk * K)])
            pltpu.sync_copy(oi_v, idx.at[pl.ds(rowbase * K, rblk * K)])
            return 0

        lax.fori_loop(0, rpw // rblk, do_block, 0)

    return knn_sc


def kernel(xyz1, xyz2):
    b, n1, _ = xyz1.shape
    n2 = xyz2.shape[1]
    q = 512
    xyz2t = jnp.transpose(xyz2, (0, 2, 1))
    r1 = _rtne_bf16(xyz1)
    r2t = _rtne_bf16(xyz2t)

    stage_a = pl.pallas_call(
        functools.partial(_stage_a, n2=n2),
        grid=(1, n1 // q),
        in_specs=[
            pl.BlockSpec((1, q, 3), lambda i, j: (i, j, 0)),
            pl.BlockSpec((1, 3, n2), lambda i, j: (i, 0, 0)),
            pl.BlockSpec((1, q, 3), lambda i, j: (i, j, 0)),
            pl.BlockSpec((1, 3, n2), lambda i, j: (i, 0, 0)),
        ],
        out_specs=[
            pl.BlockSpec((1, q, NSEG), lambda i, j: (i, j, 0)),
            pl.BlockSpec((1, q, NSEG), lambda i, j: (i, j, 0)),
            pl.BlockSpec((1, q, 1), lambda i, j: (i, j, 0)),
            pl.BlockSpec((1, 1, n2), lambda i, j: (i, 0, 0)),
        ],
        out_shape=[
            jax.ShapeDtypeStruct((1, n1, NSEG), jnp.float32),
            jax.ShapeDtypeStruct((1, n1, NSEG), jnp.int32),
            jax.ShapeDtypeStruct((1, n1, 1), jnp.float32),
            jax.ShapeDtypeStruct((1, 1, n2), jnp.float32),
        ],
    )

    sc_topk = _make_sc_topk(n1, n1 // 32, 128, n2, n1)
    dists, idxs = [], []
    for bi in range(b):
        segmin, segarg, sq1, sq2 = stage_a(
            xyz1[bi:bi + 1], xyz2t[bi:bi + 1],
            r1[bi:bi + 1], r2t[bi:bi + 1])
        kpack = jnp.concatenate(
            [r2t[bi:bi + 1], sq2], axis=1).reshape(4 * n2)
        qpack = jnp.concatenate(
            [r1[bi:bi + 1], sq1], axis=2).reshape(n1 * 4)
        d_flat, i_flat = sc_topk(segmin.reshape(n1 * NSEG),
                                 segarg.reshape(n1 * NSEG),
                                 kpack, qpack)
        dists.append(d_flat.reshape(1, n1, K))
        idxs.append(i_flat.reshape(1, n1, K))
    dist = jnp.concatenate(dists, axis=0)
    idx = jnp.concatenate(idxs, axis=0)
    mask = dist <= RADIUS
    return dist, idx, mask
